# baseline (speedup 1.0000x reference)
import functools

import jax
import jax.numpy as jnp
from jax import lax
from jax.experimental import pallas as pl
from jax.experimental.pallas import tpu as pltpu
from jax.experimental.pallas import tpu_sc as plsc

B = 1024
S = 200
D = 64
VOCAB = 100000
NC = 2
NS = 16
NW = NC * NS
BT = B // 128
UNITS = S * BT
UPW = UNITS // NW
LANES = 16


def _mesh():
    return plsc.VectorSubcoreMesh(
        core_axis_name="c", subcore_axis_name="s", num_cores=NC, num_subcores=NS
    )


@functools.partial(
    pl.kernel,
    out_type=jax.ShapeDtypeStruct((S, D // 8, BT, 8, 128), jnp.float32),
    mesh=_mesh(),
    scratch_types=[
        pltpu.VMEM((2, BT, 8, 128), jnp.int32),
        pltpu.VMEM((S, D), jnp.float32),
        pltpu.VMEM((128, D), jnp.float32),
        pltpu.VMEM((128, D), jnp.float32),
        pltpu.VMEM((D // 8, 8, 129), jnp.float32),
        pltpu.VMEM((D // 8, 8, 129), jnp.float32),
        pltpu.SemaphoreType.DMA((2,)),
        pltpu.SemaphoreType.DMA((2,)),
    ],
    compiler_params=pltpu.CompilerParams(
        use_tc_tiling_on_sc=False, needs_layout_passes=False
    ),
)
def _embed(ids_hbm, tok_hbm, pos_hbm, out_hbm, ids_v, pos_v, rows0, rows1,
           ob0, ob1, sem_g, sem_s):
    cid = lax.axis_index("c")
    sid = lax.axis_index("s")
    wid = sid * NC + cid
    u0 = wid * UPW
    s_first = u0 // BT
    st_a = s_first // 8
    st_b = jnp.minimum(st_a + 1, S // 8 - 1)

    rows = (rows0, rows1)
    obuf = (ob0, ob1)

    pltpu.sync_copy(ids_hbm.at[st_a], ids_v.at[0])
    pltpu.sync_copy(ids_hbm.at[st_b], ids_v.at[1])
    pltpu.sync_copy(pos_hbm.at[pl.ds(0, S)], pos_v)

    lane = lax.iota(jnp.int32, LANES)
    dtv = [(lane + 16 * k) // 8 for k in range(4)]
    d8v = [(lane + 16 * k) % 8 for k in range(4)]

    def gather(gu, p):
        s = (u0 + gu) // BT
        bt = (u0 + gu) % BT
        return pltpu.make_async_copy(
            tok_hbm.at[ids_v.at[s // 8 - st_a, bt, s % 8]], rows[p], sem_g.at[p]
        )

    def store(gu, p):
        s = (u0 + gu) // BT
        bt = (u0 + gu) % BT
        return pltpu.make_async_copy(
            obuf[p].at[:, :, pl.ds(0, 128)], out_hbm.at[s, :, bt], sem_s.at[p]
        )

    gather(0, 0).start()
    gather(1, 1).start()

    def t_body(t, carry):
        for u in range(2):
            gu = 2 * t + u
            s = (u0 + gu) // BT

            @pl.when(gu >= 2)
            def _():
                store(gu - 2, u).wait()

            gather(gu, u).wait()

            prow = [pos_v[s, pl.ds(16 * k, LANES)] for k in range(4)]

            @plsc.parallel_loop(0, 128, unroll=8)
            def _(b):
                b_splat = jnp.full((LANES,), b, jnp.int32)
                for k in range(4):
                    v = rows[u][b, pl.ds(16 * k, LANES)] + prow[k]
                    plsc.store_scatter(obuf[u], [dtv[k], d8v[k], b_splat], v)

            store(gu, u).start()

            @pl.when(gu + 2 < UPW)
            def _():
                gather(gu + 2, u).start()
        return carry

    lax.fori_loop(0, UPW // 2, t_body, 0)

    store(UPW - 2, 0).wait()
    store(UPW - 1, 1).wait()


def kernel(input_ids, token_table, position_table):
    ids4 = input_ids.astype(jnp.int32).T.reshape(S // 8, 8, B // 128, 128)
    ids4 = ids4.transpose(0, 2, 1, 3)
    out5 = _embed(ids4, token_table, position_table)
    return out5.transpose(2, 4, 0, 1, 3).reshape(B, S, D)


# speedup vs baseline: 1.0760x; 1.0760x over previous
import functools

import jax
import jax.numpy as jnp
from jax import lax
from jax.experimental import pallas as pl
from jax.experimental.pallas import tpu as pltpu
from jax.experimental.pallas import tpu_sc as plsc

B = 1024
S = 200
D = 64
VOCAB = 100000
NC = 2
NS = 16
NW = NC * NS
BT = B // 128
UNITS = S * BT
UPW = UNITS // NW
LANES = 16
NR = 5
AHEAD = NR - 1
NT = UPW // NR


def _mesh():
    return plsc.VectorSubcoreMesh(
        core_axis_name="c", subcore_axis_name="s", num_cores=NC, num_subcores=NS
    )


@functools.partial(
    pl.kernel,
    out_type=jax.ShapeDtypeStruct((S, D // 8, BT, 8, 128), jnp.float32),
    mesh=_mesh(),
    scratch_types=[
        pltpu.VMEM((2, BT, 8, 128), jnp.int32),
        pltpu.VMEM((S, D), jnp.float32),
    ]
    + [pltpu.VMEM((128, D), jnp.float32) for _ in range(NR)]
    + [pltpu.VMEM((D // 8, 8, 129), jnp.float32) for _ in range(NR)]
    + [
        pltpu.SemaphoreType.DMA((NR,)),
        pltpu.SemaphoreType.DMA((NR,)),
    ],
    compiler_params=pltpu.CompilerParams(
        use_tc_tiling_on_sc=False, needs_layout_passes=False
    ),
)
def _embed(ids_hbm, tok_hbm, pos_hbm, out_hbm, ids_v, pos_v, *bufs):
    rows = bufs[:NR]
    obuf = bufs[NR:2 * NR]
    sem_g, sem_s = bufs[2 * NR], bufs[2 * NR + 1]

    cid = lax.axis_index("c")
    sid = lax.axis_index("s")
    wid = sid * NC + cid
    u0 = wid * UPW
    s_first = u0 // BT
    st_a = s_first // 8
    st_b = jnp.minimum(st_a + 1, S // 8 - 1)

    pltpu.sync_copy(ids_hbm.at[st_a], ids_v.at[0])
    pltpu.sync_copy(ids_hbm.at[st_b], ids_v.at[1])
    pltpu.sync_copy(pos_hbm.at[pl.ds(0, S)], pos_v)

    lane = lax.iota(jnp.int32, LANES)
    dtv = [(lane + 16 * k) // 8 for k in range(4)]
    d8v = [(lane + 16 * k) % 8 for k in range(4)]

    def gather(gu, p):
        s = (u0 + gu) // BT
        bt = (u0 + gu) % BT
        return pltpu.make_async_copy(
            tok_hbm.at[ids_v.at[s // 8 - st_a, bt, s % 8]], rows[p], sem_g.at[p]
        )

    def store(gu, p):
        s = (u0 + gu) // BT
        bt = (u0 + gu) % BT
        return pltpu.make_async_copy(
            obuf[p].at[:, :, pl.ds(0, 128)], out_hbm.at[s, :, bt], sem_s.at[p]
        )

    for j in range(AHEAD):
        gather(j, j).start()

    def t_body(t, carry):
        for u in range(NR):
            gu = NR * t + u
            s = (u0 + gu) // BT

            @pl.when(gu >= NR)
            def _():
                store(gu - NR, u).wait()

            gather(gu, u).wait()

            prow = [pos_v[s, pl.ds(16 * k, LANES)] for k in range(4)]

            @plsc.parallel_loop(0, 128, unroll=8)
            def _(b):
                b_splat = jnp.full((LANES,), b, jnp.int32)
                for k in range(4):
                    v = rows[u][b, pl.ds(16 * k, LANES)] + prow[k]
                    plsc.store_scatter(obuf[u], [dtv[k], d8v[k], b_splat], v)

            store(gu, u).start()

            @pl.when(gu + AHEAD < UPW)
            def _():
                gather(gu + AHEAD, (u + AHEAD) % NR).start()
        return carry

    lax.fori_loop(0, NT, t_body, 0)

    for b in range(NR):
        store(UPW - NR + b, b).wait()


def kernel(input_ids, token_table, position_table):
    ids4 = input_ids.astype(jnp.int32).T.reshape(S // 8, 8, B // 128, 128)
    ids4 = ids4.transpose(0, 2, 1, 3)
    out5 = _embed(ids4, token_table, position_table)
    return out5.transpose(2, 4, 0, 1, 3).reshape(B, S, D)


# speedup vs baseline: 1.0809x; 1.0045x over previous
import functools

import jax
import jax.numpy as jnp
from jax import lax
from jax.experimental import pallas as pl
from jax.experimental.pallas import tpu as pltpu
from jax.experimental.pallas import tpu_sc as plsc

B = 1024
S = 200
D = 64
VOCAB = 100000
NC = 2
NS = 16
NW = NC * NS
BT = B // 128
UNITS = S * BT
UPW = UNITS // NW
LANES = 16
NR = 5
AHEAD = NR - 1
NT = UPW // NR


def _mesh():
    return plsc.VectorSubcoreMesh(
        core_axis_name="c", subcore_axis_name="s", num_cores=NC, num_subcores=NS
    )


@functools.partial(
    pl.kernel,
    out_type=jax.ShapeDtypeStruct((S, D // 8, BT, 8, 128), jnp.float32),
    mesh=_mesh(),
    scratch_types=[
        pltpu.VMEM((2, BT, 8, 128), jnp.int32),
        pltpu.VMEM((S, D), jnp.float32),
    ]
    + [pltpu.VMEM((128, D), jnp.float32) for _ in range(NR)]
    + [pltpu.VMEM((D // 8, 8, 129), jnp.float32) for _ in range(NR)]
    + [
        pltpu.SemaphoreType.DMA((NR,)),
        pltpu.SemaphoreType.DMA((NR,)),
    ],
    compiler_params=pltpu.CompilerParams(
        use_tc_tiling_on_sc=False, needs_layout_passes=False
    ),
)
def _embed(ids_hbm, tok_hbm, pos_hbm, out_hbm, ids_v, pos_v, *bufs):
    rows = bufs[:NR]
    obuf = bufs[NR:2 * NR]
    sem_g, sem_s = bufs[2 * NR], bufs[2 * NR + 1]

    cid = lax.axis_index("c")
    sid = lax.axis_index("s")
    wid = sid * NC + cid
    u0 = wid * UPW
    s_first = u0 // BT
    st_a = s_first // 8
    st_b = jnp.minimum(st_a + 1, S // 8 - 1)

    pltpu.sync_copy(ids_hbm.at[st_a], ids_v.at[0])
    pltpu.sync_copy(ids_hbm.at[st_b], ids_v.at[1])
    pltpu.sync_copy(pos_hbm.at[pl.ds(0, S)], pos_v)

    lane = lax.iota(jnp.int32, LANES)
    dtv = [(lane + 16 * k) // 8 for k in range(4)]
    d8v = [(lane + 16 * k) % 8 for k in range(4)]

    def gather(gu, p):
        s = (u0 + gu) // BT
        bt = (u0 + gu) % BT
        return pltpu.make_async_copy(
            tok_hbm.at[ids_v.at[s // 8 - st_a, bt, s % 8]], rows[p], sem_g.at[p]
        )

    def store(gu, p):
        s = (u0 + gu) // BT
        bt = (u0 + gu) % BT
        return pltpu.make_async_copy(
            obuf[p].at[:, :, pl.ds(0, 128)], out_hbm.at[s, :, bt], sem_s.at[p]
        )

    for j in range(AHEAD):
        gather(j, j).start()

    def t_body(t, carry):
        for u in range(NR):
            gu = NR * t + u
            s = (u0 + gu) // BT

            @pl.when(gu >= NR)
            def _():
                store(gu - NR, u).wait()

            gather(gu, u).wait()

            @pl.when(gu + AHEAD < UPW)
            def _():
                gather(gu + AHEAD, (u + AHEAD) % NR).start()

            prow = [pos_v[s, pl.ds(16 * k, LANES)] for k in range(4)]

            @plsc.parallel_loop(0, 128, unroll=8)
            def _(b):
                b_splat = jnp.full((LANES,), b, jnp.int32)
                for k in range(4):
                    v = rows[u][b, pl.ds(16 * k, LANES)] + prow[k]
                    plsc.store_scatter(obuf[u], [dtv[k], d8v[k], b_splat], v)

            store(gu, u).start()
        return carry

    lax.fori_loop(0, NT, t_body, 0)

    for b in range(NR):
        store(UPW - NR + b, b).wait()


def kernel(input_ids, token_table, position_table):
    ids4 = input_ids.astype(jnp.int32).T.reshape(S // 8, 8, B // 128, 128)
    ids4 = ids4.transpose(0, 2, 1, 3)
    out5 = _embed(ids4, token_table, position_table)
    return out5.transpose(2, 4, 0, 1, 3).reshape(B, S, D)


# speedup vs baseline: 1.1111x; 1.0279x over previous
import functools

import jax
import jax.numpy as jnp
from jax import lax
from jax.experimental import pallas as pl
from jax.experimental.pallas import tpu as pltpu
from jax.experimental.pallas import tpu_sc as plsc

B = 1024
S = 200
D = 64
VOCAB = 100000
NC = 2
NS = 16
NW = NC * NS
BT = B // 128
UNITS = S * BT
UPW = UNITS // NW
LANES = 16
NR = 5
AHEAD = NR - 1
NT = UPW // NR


def _mesh():
    return plsc.VectorSubcoreMesh(
        core_axis_name="c", subcore_axis_name="s", num_cores=NC, num_subcores=NS
    )


@functools.partial(
    pl.kernel,
    out_type=jax.ShapeDtypeStruct((S, D // 8, BT, 8, 128), jnp.float32),
    mesh=_mesh(),
    scratch_types=[
        pltpu.VMEM((2, BT, 8, 128), jnp.int32),
        pltpu.VMEM((S, D), jnp.float32),
    ]
    + [pltpu.VMEM((128, D), jnp.float32) for _ in range(NR)]
    + [pltpu.VMEM((D // 8, 8, 129), jnp.float32) for _ in range(NR)]
    + [
        pltpu.SemaphoreType.DMA((NR,)),
        pltpu.SemaphoreType.DMA((NR,)),
    ],
    compiler_params=pltpu.CompilerParams(
        use_tc_tiling_on_sc=False, needs_layout_passes=False
    ),
)
def _embed(ids_hbm, tok_hbm, pos_hbm, out_hbm, ids_v, pos_v, *bufs):
    rows = bufs[:NR]
    obuf = bufs[NR:2 * NR]
    sem_g, sem_s = bufs[2 * NR], bufs[2 * NR + 1]

    cid = lax.axis_index("c")
    sid = lax.axis_index("s")
    wid = sid * NC + cid
    u0 = wid * UPW
    s_first = u0 // BT
    st_a = s_first // 8
    st_b = jnp.minimum(st_a + 1, S // 8 - 1)

    pltpu.sync_copy(ids_hbm.at[st_a], ids_v.at[0])
    pltpu.sync_copy(ids_hbm.at[st_b], ids_v.at[1])
    pltpu.sync_copy(pos_hbm.at[pl.ds(0, S)], pos_v)

    lane = lax.iota(jnp.int32, LANES)
    dtv = [(lane + 16 * k) // 8 for k in range(4)]
    d8v = [(lane + 16 * k) % 8 for k in range(4)]

    def gather(gu, p):
        s = (u0 + gu) // BT
        bt = (u0 + gu) % BT
        return pltpu.make_async_copy(
            tok_hbm.at[ids_v.at[s // 8 - st_a, bt, s % 8]], rows[p], sem_g.at[p]
        )

    def store(gu, p):
        s = (u0 + gu) // BT
        bt = (u0 + gu) % BT
        return pltpu.make_async_copy(
            obuf[p].at[:, :, pl.ds(0, 128)], out_hbm.at[s, :, bt], sem_s.at[p]
        )

    for j in range(AHEAD):
        gather(j, j).start()

    def t_body(t, carry):
        for u in range(NR):
            gu = NR * t + u
            s = (u0 + gu) // BT

            @pl.when(gu >= NR)
            def _():
                store(gu - NR, u).wait()

            gather(gu, u).wait()

            @pl.when(gu + AHEAD < UPW)
            def _():
                gather(gu + AHEAD, (u + AHEAD) % NR).start()

            prow = [pos_v[s, pl.ds(16 * k, LANES)] for k in range(4)]

            @plsc.parallel_loop(0, 128, unroll=16)
            def _(b):
                b_splat = jnp.full((LANES,), b, jnp.int32)
                for k in range(4):
                    v = rows[u][b, pl.ds(16 * k, LANES)] + prow[k]
                    plsc.store_scatter(obuf[u], [dtv[k], d8v[k], b_splat], v)

            store(gu, u).start()
        return carry

    lax.fori_loop(0, NT, t_body, 0)

    for b in range(NR):
        store(UPW - NR + b, b).wait()


def kernel(input_ids, token_table, position_table):
    ids4 = input_ids.astype(jnp.int32).T.reshape(S // 8, 8, B // 128, 128)
    ids4 = ids4.transpose(0, 2, 1, 3)
    out5 = _embed(ids4, token_table, position_table)
    return out5.transpose(2, 4, 0, 1, 3).reshape(B, S, D)
